# Initial kernel scaffold; baseline (speedup 1.0000x reference)
import jax
import jax.numpy as jnp
from jax.experimental import pallas as pl


def kernel(X, vertex, edges, X0, W1, b1, W2, b2):
    raise NotImplementedError("write your pallas kernel here")


# speedup vs baseline: 4.1143x; 4.1143x over previous
import functools

import jax
import jax.numpy as jnp
from jax import lax
from jax.experimental import pallas as pl
from jax.experimental.pallas import tpu as pltpu
from jax.experimental.pallas import tpu_sc as plsc

ALPHA = 0.5
BETA = 1.0
M_EDGES = 80000


def _sc_two_hop(N, E, M, D, *, L=16, NC=2, NS=16, BLK=80, K=10, ZR=1000,
                interpret=False):
    S = D // L
    SPC = S // NC
    SB = K * BLK
    NSB = E // (NS * SB)
    STRIPE_M = M // NS
    STRIPE_N = N // NS
    ZCOP = STRIPE_M // ZR
    assert S * L == D and SPC * NC == S
    assert NS * SB * NSB == E
    assert ZCOP * ZR == STRIPE_M and ZR >= STRIPE_N and STRIPE_N * NS == N
    assert BLK % L == 0 and BLK <= 128

    mesh = plsc.VectorSubcoreMesh(core_axis_name="core", subcore_axis_name="sub",
                                  num_cores=NC, num_subcores=NS)

    @functools.partial(
        pl.kernel,
        out_type=jax.ShapeDtypeStruct((N, S, L), jnp.float32),
        mesh=mesh,
        interpret=interpret,
        compiler_params=pltpu.CompilerParams(use_tc_tiling_on_sc=False),
        scratch_types=[
            pltpu.HBM((NC * M, L), jnp.float32),
            pltpu.VMEM_SHARED((M, L), jnp.float32),
            pltpu.VMEM_SHARED((N, L), jnp.float32),
            pltpu.VMEM((K, BLK), jnp.int32),
            pltpu.VMEM((K, BLK), jnp.int32),
            pltpu.VMEM((K, BLK), jnp.int32),
            pltpu.VMEM((K, BLK, L), jnp.float32),
            pltpu.VMEM((ZR, L), jnp.float32),
            pltpu.SemaphoreType.DMA,
            pltpu.SemaphoreType.DMA,
        ],
    )
    def sc_pass(x_sm, vtx2d, edg2d, xv_out, xe_slab, acc1, acc2,
                vblk, eblk, gidx, rows, zbuf, gsem, ssem):
        c = lax.axis_index("core")
        w = lax.axis_index("sub")

        @pl.loop(0, ZR)
        def _zero(i):
            zbuf[i, :] = jnp.zeros((L,), jnp.float32)

        def stream_blocks(idx_adj_src, idx_adj_base, scat_src, table, acc):
            @pl.loop(0, NSB)
            def _sb(t):
                row0 = (w * NSB + t) * K
                pltpu.sync_copy(vtx2d.at[pl.ds(row0, K)], vblk)
                pltpu.sync_copy(edg2d.at[pl.ds(row0, K)], eblk)
                asrc = vblk if idx_adj_src == 0 else eblk
                ssrc = eblk if idx_adj_src == 0 else vblk
                for k in range(K):
                    for g in range(BLK // L):
                        sl = pl.ds(g * L, L)
                        gidx[k, sl] = asrc[k, sl] + idx_adj_base
                cops = [pltpu.async_copy(table.at[gidx.at[k]], rows.at[k], gsem)
                        for k in range(K)]
                for cop in cops:
                    cop.wait()
                cops = [pltpu.async_copy(rows.at[k], acc.at[ssrc.at[k]], ssem,
                                         add=True)
                        for k in range(K)]
                for cop in cops:
                    cop.wait()

        for j in range(SPC):
            s = c * SPC + j
            for z in range(ZCOP):
                pltpu.sync_copy(zbuf.at[pl.ds(0, ZR)],
                                acc1.at[pl.ds(w * STRIPE_M + z * ZR, ZR)])
            pltpu.sync_copy(zbuf.at[pl.ds(0, STRIPE_N)],
                            acc2.at[pl.ds(w * STRIPE_N, STRIPE_N)])
            plsc.subcore_barrier()
            stream_blocks(0, s * N, None, x_sm, acc1)
            plsc.subcore_barrier()
            pltpu.sync_copy(acc1.at[pl.ds(w * STRIPE_M, STRIPE_M)],
                            xe_slab.at[pl.ds(c * M + w * STRIPE_M, STRIPE_M)])
            plsc.subcore_barrier()
            stream_blocks(1, c * M, None, xe_slab, acc2)
            plsc.subcore_barrier()
            pltpu.sync_copy(acc2.at[pl.ds(w * STRIPE_N, STRIPE_N)],
                            xv_out.at[pl.ds(w * STRIPE_N, STRIPE_N), s, :])

    return sc_pass


def _mlp_tc(xv, x0, w1, b1, w2, b2, *, interpret=False):
    N, D = xv.shape
    R = 1000 if N % 1000 == 0 else N
    grid = N // R

    def body(xv_ref, x0_ref, w1_ref, b1_ref, w2_ref, b2_ref, o_ref):
        xi = (1.0 - ALPHA) * xv_ref[...] + ALPHA * x0_ref[...]
        h = jnp.maximum(
            jnp.dot(xi, w1_ref[...], preferred_element_type=jnp.float32)
            + b1_ref[...], 0.0)
        o = (jnp.dot(h, w2_ref[...], preferred_element_type=jnp.float32)
             + b2_ref[...])
        o_ref[...] = (1.0 - BETA) * xi + BETA * o

    return pl.pallas_call(
        body,
        grid=(grid,),
        in_specs=[
            pl.BlockSpec((R, D), lambda i: (i, 0)),
            pl.BlockSpec((R, D), lambda i: (i, 0)),
            pl.BlockSpec((D, D), lambda i: (0, 0)),
            pl.BlockSpec((1, D), lambda i: (0, 0)),
            pl.BlockSpec((D, D), lambda i: (0, 0)),
            pl.BlockSpec((1, D), lambda i: (0, 0)),
        ],
        out_specs=pl.BlockSpec((R, D), lambda i: (i, 0)),
        out_shape=jax.ShapeDtypeStruct((N, D), jnp.float32),
        interpret=interpret,
    )(xv, x0, w1, b1.reshape(1, D), w2, b2.reshape(1, D))


def kernel(X, vertex, edges, X0, W1, b1, W2, b2):
    N, D = X.shape
    E = vertex.shape[0]
    M = M_EDGES
    L = 16
    S = D // L
    BLK = 80
    x_sm = X.reshape(N, S, L).transpose(1, 0, 2).reshape(S * N, L)
    vtx2d = vertex.reshape(E // BLK, BLK)
    edg2d = edges.reshape(E // BLK, BLK)
    sc = _sc_two_hop(N, E, M, D, BLK=BLK)
    xv = sc(x_sm, vtx2d, edg2d).reshape(N, D)
    return _mlp_tc(xv, X0, W1, b1, W2, b2)


# speedup vs baseline: 5.2570x; 1.2777x over previous
import functools

import jax
import jax.numpy as jnp
from jax import lax
from jax.experimental import pallas as pl
from jax.experimental.pallas import tpu as pltpu
from jax.experimental.pallas import tpu_sc as plsc

ALPHA = 0.5
BETA = 1.0
M_EDGES = 80000


def _sc_two_hop(N, E, M, D, *, L=16, NC=2, NS=16, BLK=80, K=10, ZR=1000,
                interpret=False):
    S = D // L
    SPC = S // NC
    SB = K * BLK
    NSB = E // (NS * SB)
    STRIPE_M = M // NS
    STRIPE_N = N // NS
    ZCOP = STRIPE_M // ZR
    assert S * L == D and SPC * NC == S
    assert NS * SB * NSB == E
    assert ZCOP * ZR == STRIPE_M and ZR >= STRIPE_N and STRIPE_N * NS == N
    assert BLK % L == 0 and BLK <= 128

    mesh = plsc.VectorSubcoreMesh(core_axis_name="core", subcore_axis_name="sub",
                                  num_cores=NC, num_subcores=NS)

    @functools.partial(
        pl.kernel,
        out_type=jax.ShapeDtypeStruct((N, S, L), jnp.float32),
        mesh=mesh,
        interpret=interpret,
        compiler_params=pltpu.CompilerParams(use_tc_tiling_on_sc=False),
        scratch_types=[
            pltpu.VMEM_SHARED((N, L), jnp.float32),
            pltpu.VMEM_SHARED((M, L), jnp.float32),
            pltpu.VMEM_SHARED((N, L), jnp.float32),
            pltpu.VMEM((K, BLK), jnp.int32),
            pltpu.VMEM((K, BLK), jnp.int32),
            pltpu.VMEM((K, BLK, L), jnp.float32),
            pltpu.VMEM((ZR, L), jnp.float32),
            pltpu.SemaphoreType.DMA,
            pltpu.SemaphoreType.DMA,
        ],
    )
    def sc_pass(x_sm, vtx2d, edg2d, xv_out, xslab, acc1, acc2,
                vblk, eblk, rows, zbuf, gsem, ssem):
        c = lax.axis_index("core")
        w = lax.axis_index("sub")

        @pl.loop(0, ZR)
        def _zero(i):
            zbuf[i, :] = jnp.zeros((L,), jnp.float32)

        def stream_blocks(table, gather_is_vertex, acc):
            @pl.loop(0, NSB)
            def _sb(t):
                row0 = (w * NSB + t) * K
                pltpu.sync_copy(vtx2d.at[pl.ds(row0, K)], vblk)
                pltpu.sync_copy(edg2d.at[pl.ds(row0, K)], eblk)
                gsrc = vblk if gather_is_vertex else eblk
                ssrc = eblk if gather_is_vertex else vblk
                cops = [pltpu.async_copy(table.at[gsrc.at[k]], rows.at[k], gsem)
                        for k in range(K)]
                for cop in cops:
                    cop.wait()
                cops = [pltpu.async_copy(rows.at[k], acc.at[ssrc.at[k]], ssem,
                                         add=True)
                        for k in range(K)]
                for cop in cops:
                    cop.wait()

        for j in range(SPC):
            s = c * SPC + j
            pltpu.sync_copy(x_sm.at[pl.ds(s * N + w * STRIPE_N, STRIPE_N)],
                            xslab.at[pl.ds(w * STRIPE_N, STRIPE_N)])
            for z in range(ZCOP):
                pltpu.sync_copy(zbuf.at[pl.ds(0, ZR)],
                                acc1.at[pl.ds(w * STRIPE_M + z * ZR, ZR)])
            pltpu.sync_copy(zbuf.at[pl.ds(0, STRIPE_N)],
                            acc2.at[pl.ds(w * STRIPE_N, STRIPE_N)])
            plsc.subcore_barrier()
            stream_blocks(xslab, True, acc1)
            plsc.subcore_barrier()
            stream_blocks(acc1, False, acc2)
            plsc.subcore_barrier()
            pltpu.sync_copy(acc2.at[pl.ds(w * STRIPE_N, STRIPE_N)],
                            xv_out.at[pl.ds(w * STRIPE_N, STRIPE_N), s, :])

    return sc_pass


def _mlp_tc(xv, x0, w1, b1, w2, b2, *, interpret=False):
    N, D = xv.shape
    R = 1000 if N % 1000 == 0 else N
    grid = N // R

    def body(xv_ref, x0_ref, w1_ref, b1_ref, w2_ref, b2_ref, o_ref):
        xi = (1.0 - ALPHA) * xv_ref[...] + ALPHA * x0_ref[...]
        h = jnp.maximum(
            jnp.dot(xi, w1_ref[...], preferred_element_type=jnp.float32)
            + b1_ref[...], 0.0)
        o = (jnp.dot(h, w2_ref[...], preferred_element_type=jnp.float32)
             + b2_ref[...])
        o_ref[...] = (1.0 - BETA) * xi + BETA * o

    return pl.pallas_call(
        body,
        grid=(grid,),
        in_specs=[
            pl.BlockSpec((R, D), lambda i: (i, 0)),
            pl.BlockSpec((R, D), lambda i: (i, 0)),
            pl.BlockSpec((D, D), lambda i: (0, 0)),
            pl.BlockSpec((1, D), lambda i: (0, 0)),
            pl.BlockSpec((D, D), lambda i: (0, 0)),
            pl.BlockSpec((1, D), lambda i: (0, 0)),
        ],
        out_specs=pl.BlockSpec((R, D), lambda i: (i, 0)),
        out_shape=jax.ShapeDtypeStruct((N, D), jnp.float32),
        interpret=interpret,
    )(xv, x0, w1, b1.reshape(1, D), w2, b2.reshape(1, D))


def kernel(X, vertex, edges, X0, W1, b1, W2, b2):
    N, D = X.shape
    E = vertex.shape[0]
    M = M_EDGES
    L = 16
    S = D // L
    BLK = 80
    x_sm = X.reshape(N, S, L).transpose(1, 0, 2).reshape(S * N, L)
    vtx2d = vertex.reshape(E // BLK, BLK)
    edg2d = edges.reshape(E // BLK, BLK)
    sc = _sc_two_hop(N, E, M, D, BLK=BLK)
    xv = sc(x_sm, vtx2d, edg2d).reshape(N, D)
    return _mlp_tc(xv, X0, W1, b1, W2, b2)


# speedup vs baseline: 7.9645x; 1.5150x over previous
import functools

import jax
import jax.numpy as jnp
from jax import lax
from jax.experimental import pallas as pl
from jax.experimental.pallas import tpu as pltpu
from jax.experimental.pallas import tpu_sc as plsc

ALPHA = 0.5
BETA = 1.0
M_EDGES = 80000


def _sc_two_hop(N, E, M, D, *, L=16, NC=2, NS=16, BLK=125, K=8, ZR=125,
                interpret=False):
    S = D // L
    SPC = S // NC
    SB = K * BLK
    NSB = E // (NS * SB)
    RPT = E // (NS * BLK)
    STRIPE_M = M // NS
    STRIPE_N = N // NS
    ZCOP = STRIPE_M // ZR
    assert S * L == D and SPC * NC == S
    assert NS * SB * NSB == E and RPT == NSB * K
    assert ZCOP * ZR == STRIPE_M and STRIPE_N % ZR == 0 and STRIPE_N * NS == N
    assert BLK <= 128

    mesh = plsc.VectorSubcoreMesh(core_axis_name="core", subcore_axis_name="sub",
                                  num_cores=NC, num_subcores=NS)

    @functools.partial(
        pl.kernel,
        out_type=jax.ShapeDtypeStruct((N, S, L), jnp.float32),
        mesh=mesh,
        interpret=interpret,
        compiler_params=pltpu.CompilerParams(use_tc_tiling_on_sc=False),
        scratch_types=[
            pltpu.VMEM_SHARED((M, L), jnp.float32),
            pltpu.VMEM_SHARED((N, L), jnp.float32),
            pltpu.VMEM((2, K, BLK), jnp.int32),
            pltpu.VMEM((2, K, BLK), jnp.int32),
            pltpu.VMEM((2, K, BLK, L), jnp.float32),
            pltpu.VMEM((ZR, L), jnp.float32),
            pltpu.SemaphoreType.DMA,
            pltpu.SemaphoreType.DMA,
            pltpu.SemaphoreType.DMA,
            pltpu.SemaphoreType.DMA,
        ],
    )
    def sc_pass(x_sm, vtx2d, edg2d, xv_out, acc1, xacc,
                vibuf, eibuf, rows, zbuf, gsem, ssem, vsem, esem):
        c = lax.axis_index("core")
        w = lax.axis_index("sub")
        row_base = w * RPT

        @pl.loop(0, ZR)
        def _zero(i):
            zbuf[i, :] = jnp.zeros((L,), jnp.float32)

        def stream_blocks(table, acc, gather_by_vertex):
            gibuf, gs = (vibuf, vsem) if gather_by_vertex else (eibuf, esem)
            sibuf, ss = (eibuf, esem) if gather_by_vertex else (vibuf, vsem)
            gsrc2d = vtx2d if gather_by_vertex else edg2d
            ssrc2d = edg2d if gather_by_vertex else vtx2d

            def load_idx(src2d, buf, sb, b, sem):
                return pltpu.async_copy(
                    src2d.at[pl.ds(row_base + sb * K, K)], buf.at[b], sem)

            def fire_g(b):
                for k in range(K):
                    pltpu.async_copy(table.at[gibuf.at[b, k]], rows.at[b, k],
                                     gsem)

            def fire_s(b):
                for k in range(K):
                    pltpu.async_copy(rows.at[b, k], acc.at[sibuf.at[b, k]],
                                     ssem, add=True)

            def drain_rows(sem, b):
                for k in range(K):
                    pltpu.make_async_copy(table.at[gibuf.at[b, k]],
                                          rows.at[b, k], sem).wait()

            def wait_idx(buf, b, sem):
                pltpu.make_async_copy(gsrc2d.at[pl.ds(0, K)], buf.at[b],
                                      sem).wait()

            for b in (0, 1):
                load_idx(gsrc2d, gibuf, b, b, gs).wait()
                load_idx(ssrc2d, sibuf, b, b, ss).wait()
                fire_g(b)

            @pl.loop(0, NSB)
            def _sb(t):
                b = lax.rem(t, 2)
                drain_rows(gsem, b)

                @pl.when(t + 2 < NSB)
                def _():
                    load_idx(gsrc2d, gibuf, t + 2, b, gs)

                @pl.when(t >= 2)
                def _():
                    wait_idx(sibuf, b, ss)

                fire_s(b)
                drain_rows(ssem, b)

                @pl.when(t + 2 < NSB)
                def _():
                    load_idx(ssrc2d, sibuf, t + 2, b, ss)
                    wait_idx(gibuf, b, gs)
                    fire_g(b)

        for j in range(SPC):
            s = c * SPC + j
            cops = [pltpu.async_copy(
                x_sm.at[pl.ds(s * N + w * STRIPE_N, STRIPE_N)],
                xacc.at[pl.ds(w * STRIPE_N, STRIPE_N)], vsem)]
            for z in range(ZCOP):
                cops.append(pltpu.async_copy(
                    zbuf.at[pl.ds(0, ZR)],
                    acc1.at[pl.ds(w * STRIPE_M + z * ZR, ZR)], vsem))
            for cop in cops:
                cop.wait()
            plsc.subcore_barrier()
            stream_blocks(xacc, acc1, True)
            plsc.subcore_barrier()
            for z in range(STRIPE_N // ZR):
                pltpu.sync_copy(zbuf.at[pl.ds(0, ZR)],
                                xacc.at[pl.ds(w * STRIPE_N + z * ZR, ZR)])
            plsc.subcore_barrier()
            stream_blocks(acc1, xacc, False)
            plsc.subcore_barrier()
            pltpu.sync_copy(xacc.at[pl.ds(w * STRIPE_N, STRIPE_N)],
                            xv_out.at[pl.ds(w * STRIPE_N, STRIPE_N), s, :])

    return sc_pass


def _mlp_tc(xv, x0, w1, b1, w2, b2, *, interpret=False):
    N, D = xv.shape
    R = 1000 if N % 1000 == 0 else N
    grid = N // R

    def body(xv_ref, x0_ref, w1_ref, b1_ref, w2_ref, b2_ref, o_ref):
        xi = (1.0 - ALPHA) * xv_ref[...] + ALPHA * x0_ref[...]
        h = jnp.maximum(
            jnp.dot(xi, w1_ref[...], preferred_element_type=jnp.float32)
            + b1_ref[...], 0.0)
        o = (jnp.dot(h, w2_ref[...], preferred_element_type=jnp.float32)
             + b2_ref[...])
        o_ref[...] = (1.0 - BETA) * xi + BETA * o

    return pl.pallas_call(
        body,
        grid=(grid,),
        in_specs=[
            pl.BlockSpec((R, D), lambda i: (i, 0)),
            pl.BlockSpec((R, D), lambda i: (i, 0)),
            pl.BlockSpec((D, D), lambda i: (0, 0)),
            pl.BlockSpec((1, D), lambda i: (0, 0)),
            pl.BlockSpec((D, D), lambda i: (0, 0)),
            pl.BlockSpec((1, D), lambda i: (0, 0)),
        ],
        out_specs=pl.BlockSpec((R, D), lambda i: (i, 0)),
        out_shape=jax.ShapeDtypeStruct((N, D), jnp.float32),
        interpret=interpret,
    )(xv, x0, w1, b1.reshape(1, D), w2, b2.reshape(1, D))


def kernel(X, vertex, edges, X0, W1, b1, W2, b2):
    N, D = X.shape
    E = vertex.shape[0]
    M = M_EDGES
    L = 16
    S = D // L
    BLK = 125
    x_sm = X.reshape(N, S, L).transpose(1, 0, 2).reshape(S * N, L)
    vtx2d = vertex.reshape(E // BLK, BLK)
    edg2d = edges.reshape(E // BLK, BLK)
    sc = _sc_two_hop(N, E, M, D, BLK=BLK)
    xv = sc(x_sm, vtx2d, edg2d).reshape(N, D)
    return _mlp_tc(xv, X0, W1, b1, W2, b2)


# speedup vs baseline: 10.2262x; 1.2840x over previous
import functools

import jax
import jax.numpy as jnp
from jax import lax
from jax.experimental import pallas as pl
from jax.experimental.pallas import tpu as pltpu
from jax.experimental.pallas import tpu_sc as plsc

ALPHA = 0.5
BETA = 1.0
M_EDGES = 80000


def _sc_two_hop(N, E, M, D, *, L=16, NC=2, NS=16, BLK=125, K=8, ZR=125,
                interpret=False):
    S = D // L
    SPC = S // NC
    SB = K * BLK
    NSB = E // (NS * SB)
    RPT = E // (NS * BLK)
    STRIPE_M = M // NS
    STRIPE_N = N // NS
    ZCOP = STRIPE_M // ZR
    assert S * L == D and SPC * NC == S
    assert NS * SB * NSB == E and RPT == NSB * K
    assert ZCOP * ZR == STRIPE_M and STRIPE_N % ZR == 0 and STRIPE_N * NS == N
    assert BLK <= 128

    mesh = plsc.VectorSubcoreMesh(core_axis_name="core", subcore_axis_name="sub",
                                  num_cores=NC, num_subcores=NS)

    @functools.partial(
        pl.kernel,
        out_type=jax.ShapeDtypeStruct((N, D), jnp.float32),
        mesh=mesh,
        interpret=interpret,
        compiler_params=pltpu.CompilerParams(use_tc_tiling_on_sc=False),
        scratch_types=[
            pltpu.VMEM_SHARED((M, L), jnp.float32),
            pltpu.VMEM_SHARED((N, L), jnp.float32),
            pltpu.VMEM((2, K, BLK), jnp.int32),
            pltpu.VMEM((2, K, BLK), jnp.int32),
            pltpu.VMEM((2, K, BLK, L), jnp.float32),
            pltpu.VMEM((ZR, L), jnp.float32),
            pltpu.SemaphoreType.DMA,
            pltpu.SemaphoreType.DMA,
            pltpu.SemaphoreType.DMA,
            pltpu.SemaphoreType.DMA,
        ],
    )
    def sc_pass(x_nat, vtx2d, edg2d, xv_out, acc1, xacc,
                vibuf, eibuf, rows, zbuf, gsem, ssem, vsem, esem):
        c = lax.axis_index("core")
        w = lax.axis_index("sub")
        row_base = w * RPT

        @pl.loop(0, ZR)
        def _zero(i):
            zbuf[i, :] = jnp.zeros((L,), jnp.float32)

        def stream_blocks(table, acc, gather_by_vertex):
            gibuf, gs = (vibuf, vsem) if gather_by_vertex else (eibuf, esem)
            sibuf, ss = (eibuf, esem) if gather_by_vertex else (vibuf, vsem)
            gsrc2d = vtx2d if gather_by_vertex else edg2d
            ssrc2d = edg2d if gather_by_vertex else vtx2d

            def load_idx(src2d, buf, sb, b, sem):
                return pltpu.async_copy(
                    src2d.at[pl.ds(row_base + sb * K, K)], buf.at[b], sem)

            def fire_g(b):
                for k in range(K):
                    pltpu.async_copy(table.at[gibuf.at[b, k]], rows.at[b, k],
                                     gsem)

            def fire_s(b):
                for k in range(K):
                    pltpu.async_copy(rows.at[b, k], acc.at[sibuf.at[b, k]],
                                     ssem, add=True)

            def drain_rows(sem, b):
                for k in range(K):
                    pltpu.make_async_copy(table.at[gibuf.at[b, k]],
                                          rows.at[b, k], sem).wait()

            def wait_idx(buf, b, sem):
                pltpu.make_async_copy(gsrc2d.at[pl.ds(0, K)], buf.at[b],
                                      sem).wait()

            for b in (0, 1):
                load_idx(gsrc2d, gibuf, b, b, gs).wait()
                load_idx(ssrc2d, sibuf, b, b, ss).wait()
                fire_g(b)

            @pl.loop(0, NSB)
            def _sb(t):
                b = lax.rem(t, 2)
                drain_rows(gsem, b)

                @pl.when(t + 2 < NSB)
                def _():
                    load_idx(gsrc2d, gibuf, t + 2, b, gs)

                @pl.when(t >= 2)
                def _():
                    wait_idx(sibuf, b, ss)

                fire_s(b)
                drain_rows(ssem, b)

                @pl.when(t + 2 < NSB)
                def _():
                    load_idx(ssrc2d, sibuf, t + 2, b, ss)
                    wait_idx(gibuf, b, gs)
                    fire_g(b)

        for j in range(SPC):
            s = c * SPC + j
            cops = [pltpu.async_copy(
                x_nat.at[pl.ds(w * STRIPE_N, STRIPE_N), pl.ds(s * L, L)],
                xacc.at[pl.ds(w * STRIPE_N, STRIPE_N)], vsem)]
            for z in range(ZCOP):
                cops.append(pltpu.async_copy(
                    zbuf.at[pl.ds(0, ZR)],
                    acc1.at[pl.ds(w * STRIPE_M + z * ZR, ZR)], vsem))
            for cop in cops:
                cop.wait()
            plsc.subcore_barrier()
            stream_blocks(xacc, acc1, True)
            plsc.subcore_barrier()
            for z in range(STRIPE_N // ZR):
                pltpu.sync_copy(zbuf.at[pl.ds(0, ZR)],
                                xacc.at[pl.ds(w * STRIPE_N + z * ZR, ZR)])
            plsc.subcore_barrier()
            stream_blocks(acc1, xacc, False)
            plsc.subcore_barrier()
            pltpu.sync_copy(
                xacc.at[pl.ds(w * STRIPE_N, STRIPE_N)],
                xv_out.at[pl.ds(w * STRIPE_N, STRIPE_N), pl.ds(s * L, L)])

    return sc_pass


def _mlp_tc(xv, x0, w1, b1, w2, b2, *, interpret=False):
    N, D = xv.shape
    R = 1000 if N % 1000 == 0 else N
    grid = N // R

    def body(xv_ref, x0_ref, w1_ref, b1_ref, w2_ref, b2_ref, o_ref):
        xi = (1.0 - ALPHA) * xv_ref[...] + ALPHA * x0_ref[...]
        h = jnp.maximum(
            jnp.dot(xi, w1_ref[...], preferred_element_type=jnp.float32)
            + b1_ref[...], 0.0)
        o = (jnp.dot(h, w2_ref[...], preferred_element_type=jnp.float32)
             + b2_ref[...])
        o_ref[...] = (1.0 - BETA) * xi + BETA * o

    return pl.pallas_call(
        body,
        grid=(grid,),
        in_specs=[
            pl.BlockSpec((R, D), lambda i: (i, 0)),
            pl.BlockSpec((R, D), lambda i: (i, 0)),
            pl.BlockSpec((D, D), lambda i: (0, 0)),
            pl.BlockSpec((1, D), lambda i: (0, 0)),
            pl.BlockSpec((D, D), lambda i: (0, 0)),
            pl.BlockSpec((1, D), lambda i: (0, 0)),
        ],
        out_specs=pl.BlockSpec((R, D), lambda i: (i, 0)),
        out_shape=jax.ShapeDtypeStruct((N, D), jnp.float32),
        interpret=interpret,
    )(xv, x0, w1, b1.reshape(1, D), w2, b2.reshape(1, D))


def kernel(X, vertex, edges, X0, W1, b1, W2, b2):
    N, D = X.shape
    E = vertex.shape[0]
    M = M_EDGES
    L = 16
    S = D // L
    BLK = 125
    vtx2d = vertex.reshape(E // BLK, BLK)
    edg2d = edges.reshape(E // BLK, BLK)
    sc = _sc_two_hop(N, E, M, D, BLK=BLK)
    xv = sc(X, vtx2d, edg2d)
    return _mlp_tc(xv, X0, W1, b1, W2, b2)


# speedup vs baseline: 11.8053x; 1.1544x over previous
import functools

import jax
import jax.numpy as jnp
from jax import lax
from jax.experimental import pallas as pl
from jax.experimental.pallas import tpu as pltpu
from jax.experimental.pallas import tpu_sc as plsc

ALPHA = 0.5
BETA = 1.0
M_EDGES = 80000


def _sc_two_hop(N, E, M, D, *, L=16, NC=2, NS=16, BLK=125, K=4, RING=4,
                ZR=125, interpret=False):
    S = D // L
    SPC = S // NC
    SB = K * BLK
    NSB = E // (NS * SB)
    RPT = E // (NS * BLK)
    STRIPE_M = M // NS
    STRIPE_N = N // NS
    ZCOP = STRIPE_M // ZR
    assert S * L == D and SPC * NC == S
    assert NS * SB * NSB == E and RPT == NSB * K
    assert ZCOP * ZR == STRIPE_M and STRIPE_N % ZR == 0 and STRIPE_N * NS == N
    assert BLK <= 128 and RING == 4 and NSB >= 4

    mesh = plsc.VectorSubcoreMesh(core_axis_name="core", subcore_axis_name="sub",
                                  num_cores=NC, num_subcores=NS)

    @functools.partial(
        pl.kernel,
        out_type=jax.ShapeDtypeStruct((N, D), jnp.float32),
        mesh=mesh,
        interpret=interpret,
        compiler_params=pltpu.CompilerParams(use_tc_tiling_on_sc=False),
        scratch_types=[
            pltpu.VMEM_SHARED((M, L), jnp.float32),
            pltpu.VMEM_SHARED((N, L), jnp.float32),
            pltpu.VMEM((RING, K, BLK), jnp.int32),
            pltpu.VMEM((RING, K, BLK), jnp.int32),
            pltpu.VMEM((RING, K, BLK, L), jnp.float32),
            pltpu.VMEM((ZR, L), jnp.float32),
            pltpu.SemaphoreType.DMA,
            pltpu.SemaphoreType.DMA,
            pltpu.SemaphoreType.DMA,
            pltpu.SemaphoreType.DMA,
        ],
    )
    def sc_pass(x_nat, vtx2d, edg2d, xv_out, acc1, xacc,
                vibuf, eibuf, rows, zbuf, gsem, ssem, vsem, esem):
        c = lax.axis_index("core")
        w = lax.axis_index("sub")
        row_base = w * RPT

        @pl.loop(0, ZR)
        def _zero(i):
            zbuf[i, :] = jnp.zeros((L,), jnp.float32)

        def stream_blocks(table, acc, gather_by_vertex):
            gibuf, gs = (vibuf, vsem) if gather_by_vertex else (eibuf, esem)
            sibuf, ss = (eibuf, esem) if gather_by_vertex else (vibuf, vsem)
            gsrc2d = vtx2d if gather_by_vertex else edg2d
            ssrc2d = edg2d if gather_by_vertex else vtx2d

            def load_idx(src2d, buf, sb, slot, sem):
                return pltpu.async_copy(
                    src2d.at[pl.ds(row_base + sb * K, K)], buf.at[slot], sem)

            def fire_g(slot):
                for k in range(K):
                    pltpu.async_copy(table.at[gibuf.at[slot, k]],
                                     rows.at[slot, k], gsem)

            def fire_s(slot):
                for k in range(K):
                    pltpu.async_copy(rows.at[slot, k],
                                     acc.at[sibuf.at[slot, k]], ssem, add=True)

            def drain_rows(sem, slot):
                for k in range(K):
                    pltpu.make_async_copy(table.at[gibuf.at[slot, k]],
                                          rows.at[slot, k], sem).wait()

            def wait_idx(buf, slot, sem):
                pltpu.make_async_copy(gsrc2d.at[pl.ds(0, K)], buf.at[slot],
                                      sem).wait()

            for b in (0, 1):
                load_idx(gsrc2d, gibuf, b, b, gs).wait()
                load_idx(ssrc2d, sibuf, b, b, ss).wait()
                fire_g(b)
            load_idx(gsrc2d, gibuf, 2, 2, gs)

            @pl.loop(0, NSB)
            def _sb(t):
                b = lax.rem(t, RING)
                drain_rows(gsem, b)

                @pl.when(t >= 2)
                def _():
                    wait_idx(sibuf, b, ss)

                fire_s(b)

                @pl.when(t + 2 < NSB)
                def _():
                    b2 = lax.rem(t + 2, RING)

                    @pl.when(t >= 2)
                    def _():
                        drain_rows(ssem, b2)

                    load_idx(ssrc2d, sibuf, t + 2, b2, ss)

                    @pl.when(t + 3 < NSB)
                    def _():
                        load_idx(gsrc2d, gibuf, t + 3, lax.rem(t + 3, RING),
                                 gs)

                    wait_idx(gibuf, b2, gs)
                    fire_g(b2)

            for r in range(RING):
                drain_rows(ssem, r)

        for j in range(SPC):
            s = c * SPC + j
            cops = [pltpu.async_copy(
                x_nat.at[pl.ds(w * STRIPE_N, STRIPE_N), pl.ds(s * L, L)],
                xacc.at[pl.ds(w * STRIPE_N, STRIPE_N)], vsem)]
            for z in range(ZCOP):
                cops.append(pltpu.async_copy(
                    zbuf.at[pl.ds(0, ZR)],
                    acc1.at[pl.ds(w * STRIPE_M + z * ZR, ZR)], vsem))
            for cop in cops:
                cop.wait()
            plsc.subcore_barrier()
            stream_blocks(xacc, acc1, True)
            plsc.subcore_barrier()
            for z in range(STRIPE_N // ZR):
                pltpu.sync_copy(zbuf.at[pl.ds(0, ZR)],
                                xacc.at[pl.ds(w * STRIPE_N + z * ZR, ZR)])
            plsc.subcore_barrier()
            stream_blocks(acc1, xacc, False)
            plsc.subcore_barrier()
            pltpu.sync_copy(
                xacc.at[pl.ds(w * STRIPE_N, STRIPE_N)],
                xv_out.at[pl.ds(w * STRIPE_N, STRIPE_N), pl.ds(s * L, L)])

    return sc_pass


def _mlp_tc(xv, x0, w1, b1, w2, b2, *, interpret=False):
    N, D = xv.shape
    R = 1000 if N % 1000 == 0 else N
    grid = N // R

    def body(xv_ref, x0_ref, w1_ref, b1_ref, w2_ref, b2_ref, o_ref):
        xi = (1.0 - ALPHA) * xv_ref[...] + ALPHA * x0_ref[...]
        h = jnp.maximum(
            jnp.dot(xi, w1_ref[...], preferred_element_type=jnp.float32)
            + b1_ref[...], 0.0)
        o = (jnp.dot(h, w2_ref[...], preferred_element_type=jnp.float32)
             + b2_ref[...])
        o_ref[...] = (1.0 - BETA) * xi + BETA * o

    return pl.pallas_call(
        body,
        grid=(grid,),
        in_specs=[
            pl.BlockSpec((R, D), lambda i: (i, 0)),
            pl.BlockSpec((R, D), lambda i: (i, 0)),
            pl.BlockSpec((D, D), lambda i: (0, 0)),
            pl.BlockSpec((1, D), lambda i: (0, 0)),
            pl.BlockSpec((D, D), lambda i: (0, 0)),
            pl.BlockSpec((1, D), lambda i: (0, 0)),
        ],
        out_specs=pl.BlockSpec((R, D), lambda i: (i, 0)),
        out_shape=jax.ShapeDtypeStruct((N, D), jnp.float32),
        interpret=interpret,
    )(xv, x0, w1, b1.reshape(1, D), w2, b2.reshape(1, D))


def kernel(X, vertex, edges, X0, W1, b1, W2, b2):
    N, D = X.shape
    E = vertex.shape[0]
    M = M_EDGES
    L = 16
    S = D // L
    BLK = 125
    vtx2d = vertex.reshape(E // BLK, BLK)
    edg2d = edges.reshape(E // BLK, BLK)
    sc = _sc_two_hop(N, E, M, D, BLK=BLK)
    xv = sc(X, vtx2d, edg2d)
    return _mlp_tc(xv, X0, W1, b1, W2, b2)


# speedup vs baseline: 18.7119x; 1.5850x over previous
import functools

import jax
import jax.numpy as jnp
from jax import lax
from jax.experimental import pallas as pl
from jax.experimental.pallas import tpu as pltpu
from jax.experimental.pallas import tpu_sc as plsc

ALPHA = 0.5
BETA = 1.0
M_EDGES = 80000


def _sc_two_hop(N, E, M, D, *, L=32, DT=jnp.bfloat16, NC=2, NS=16, BLK=125,
                K=4, RING=4, ZR=125, interpret=False):
    S = D // L
    SPC = S // NC
    SB = K * BLK
    NSB = E // (NS * SB)
    RPT = E // (NS * BLK)
    STRIPE_M = M // NS
    STRIPE_N = N // NS
    ZCOP = STRIPE_M // ZR
    assert S * L == D and SPC * NC == S
    assert NS * SB * NSB == E and RPT == NSB * K
    assert ZCOP * ZR == STRIPE_M and STRIPE_N % ZR == 0 and STRIPE_N * NS == N
    assert BLK <= 128 and RING == 4 and NSB >= 4

    mesh = plsc.VectorSubcoreMesh(core_axis_name="core", subcore_axis_name="sub",
                                  num_cores=NC, num_subcores=NS)

    @functools.partial(
        pl.kernel,
        out_type=jax.ShapeDtypeStruct((N, D), DT),
        mesh=mesh,
        interpret=interpret,
        compiler_params=pltpu.CompilerParams(use_tc_tiling_on_sc=False),
        scratch_types=[
            pltpu.VMEM_SHARED((M, L), DT),
            pltpu.VMEM_SHARED((N, L), DT),
            pltpu.VMEM((RING, K, BLK), jnp.int32),
            pltpu.VMEM((RING, K, BLK), jnp.int32),
            pltpu.VMEM((RING, K, BLK, L), DT),
            pltpu.VMEM((ZR, L), DT),
            pltpu.SemaphoreType.DMA,
            pltpu.SemaphoreType.DMA,
            pltpu.SemaphoreType.DMA,
            pltpu.SemaphoreType.DMA,
        ],
    )
    def sc_pass(x_nat, vtx2d, edg2d, xv_out, acc1, xacc,
                vibuf, eibuf, rows, zbuf, gsem, ssem, vsem, esem):
        c = lax.axis_index("core")
        w = lax.axis_index("sub")
        row_base = w * RPT

        @pl.loop(0, ZR)
        def _zero(i):
            zbuf[i, :] = jnp.zeros((L,), DT)

        def stream_blocks(table, acc, gather_by_vertex):
            gibuf, gs = (vibuf, vsem) if gather_by_vertex else (eibuf, esem)
            sibuf, ss = (eibuf, esem) if gather_by_vertex else (vibuf, vsem)
            gsrc2d = vtx2d if gather_by_vertex else edg2d
            ssrc2d = edg2d if gather_by_vertex else vtx2d

            def load_idx(src2d, buf, sb, slot, sem):
                return pltpu.async_copy(
                    src2d.at[pl.ds(row_base + sb * K, K)], buf.at[slot], sem)

            def fire_g(slot):
                for k in range(K):
                    pltpu.async_copy(table.at[gibuf.at[slot, k]],
                                     rows.at[slot, k], gsem)

            def fire_s(slot):
                for k in range(K):
                    pltpu.async_copy(rows.at[slot, k],
                                     acc.at[sibuf.at[slot, k]], ssem, add=True)

            def drain_rows(sem, slot):
                for k in range(K):
                    pltpu.make_async_copy(table.at[gibuf.at[slot, k]],
                                          rows.at[slot, k], sem).wait()

            def wait_idx(buf, slot, sem):
                pltpu.make_async_copy(gsrc2d.at[pl.ds(0, K)], buf.at[slot],
                                      sem).wait()

            for b in (0, 1):
                load_idx(gsrc2d, gibuf, b, b, gs).wait()
                load_idx(ssrc2d, sibuf, b, b, ss).wait()
                fire_g(b)
            load_idx(gsrc2d, gibuf, 2, 2, gs)

            @pl.loop(0, NSB)
            def _sb(t):
                b = lax.rem(t, RING)
                drain_rows(gsem, b)

                @pl.when(t >= 2)
                def _():
                    wait_idx(sibuf, b, ss)

                fire_s(b)

                @pl.when(t + 2 < NSB)
                def _():
                    b2 = lax.rem(t + 2, RING)

                    @pl.when(t >= 2)
                    def _():
                        drain_rows(ssem, b2)

                    load_idx(ssrc2d, sibuf, t + 2, b2, ss)

                    @pl.when(t + 3 < NSB)
                    def _():
                        load_idx(gsrc2d, gibuf, t + 3, lax.rem(t + 3, RING),
                                 gs)

                    wait_idx(gibuf, b2, gs)
                    fire_g(b2)

            for r in range(RING):
                drain_rows(ssem, r)

        for j in range(SPC):
            s = c * SPC + j
            cops = [pltpu.async_copy(
                x_nat.at[pl.ds(w * STRIPE_N, STRIPE_N), pl.ds(s * L, L)],
                xacc.at[pl.ds(w * STRIPE_N, STRIPE_N)], vsem)]
            for z in range(ZCOP):
                cops.append(pltpu.async_copy(
                    zbuf.at[pl.ds(0, ZR)],
                    acc1.at[pl.ds(w * STRIPE_M + z * ZR, ZR)], vsem))
            for cop in cops:
                cop.wait()
            plsc.subcore_barrier()
            stream_blocks(xacc, acc1, True)
            plsc.subcore_barrier()
            for z in range(STRIPE_N // ZR):
                pltpu.sync_copy(zbuf.at[pl.ds(0, ZR)],
                                xacc.at[pl.ds(w * STRIPE_N + z * ZR, ZR)])
            plsc.subcore_barrier()
            stream_blocks(acc1, xacc, False)
            plsc.subcore_barrier()
            pltpu.sync_copy(
                xacc.at[pl.ds(w * STRIPE_N, STRIPE_N)],
                xv_out.at[pl.ds(w * STRIPE_N, STRIPE_N), pl.ds(s * L, L)])

    return sc_pass


def _mlp_tc(xv, x0, w1, b1, w2, b2, *, interpret=False):
    N, D = xv.shape
    R = 1000 if N % 1000 == 0 else N
    grid = N // R

    def body(xv_ref, x0_ref, w1_ref, b1_ref, w2_ref, b2_ref, o_ref):
        xi = ((1.0 - ALPHA) * xv_ref[...].astype(jnp.float32)
              + ALPHA * x0_ref[...])
        h = jnp.maximum(
            jnp.dot(xi, w1_ref[...], preferred_element_type=jnp.float32)
            + b1_ref[...], 0.0)
        o = (jnp.dot(h, w2_ref[...], preferred_element_type=jnp.float32)
             + b2_ref[...])
        o_ref[...] = (1.0 - BETA) * xi + BETA * o

    return pl.pallas_call(
        body,
        grid=(grid,),
        in_specs=[
            pl.BlockSpec((R, D), lambda i: (i, 0)),
            pl.BlockSpec((R, D), lambda i: (i, 0)),
            pl.BlockSpec((D, D), lambda i: (0, 0)),
            pl.BlockSpec((1, D), lambda i: (0, 0)),
            pl.BlockSpec((D, D), lambda i: (0, 0)),
            pl.BlockSpec((1, D), lambda i: (0, 0)),
        ],
        out_specs=pl.BlockSpec((R, D), lambda i: (i, 0)),
        out_shape=jax.ShapeDtypeStruct((N, D), jnp.float32),
        interpret=interpret,
    )(xv, x0, w1, b1.reshape(1, D), w2, b2.reshape(1, D))


def kernel(X, vertex, edges, X0, W1, b1, W2, b2):
    N, D = X.shape
    E = vertex.shape[0]
    M = M_EDGES
    BLK = 125
    vtx2d = vertex.reshape(E // BLK, BLK)
    edg2d = edges.reshape(E // BLK, BLK)
    sc = _sc_two_hop(N, E, M, D, BLK=BLK)
    xv = sc(X.astype(jnp.bfloat16), vtx2d, edg2d)
    return _mlp_tc(xv, X0, W1, b1, W2, b2)
